# baseline (speedup 1.0000x reference)
import jax
import jax.numpy as jnp
import numpy as np
from jax.experimental import pallas as pl
from jax.experimental.pallas import tpu as pltpu

B, S, D = 1, 2048, 2048
H, HD, RD = 16, 192, 64
ND = HD - RD
QLR, OLR, G = 1024, 128, 4
WINDOW = 512
EPS = 1e-6
SCALE = HD ** -0.5
NEG = -1e30

BP = 256
BQ = 512
KW = WINDOW + BQ
HPG = H // G


def _dot_nt(a, b):
    return jax.lax.dot_general(
        a.astype(jnp.bfloat16), b.astype(jnp.bfloat16),
        (((1,), (1,)), ((), ())), preferred_element_type=jnp.float32)


def _dot_nn(a, b):
    return jax.lax.dot_general(
        a.astype(jnp.bfloat16), b.astype(jnp.bfloat16),
        (((1,), (0,)), ((), ())), preferred_element_type=jnp.float32)


def _proj_kernel(x_ref, wqa_ref, qnw_ref, wqb_ref, wkv_ref, kvnw_ref,
                 cos_ref, sin_ref, qf_ref, kf_ref, v_ref):
    x = x_ref[...]
    ql = _dot_nt(x, wqa_ref[...])
    var = jnp.mean(ql * ql, axis=-1, keepdims=True)
    ql = ql * jax.lax.rsqrt(var + EPS) * qnw_ref[...]
    q = _dot_nt(ql, wqb_ref[...])

    kv = _dot_nt(x, wkv_ref[...])
    var2 = jnp.mean(kv * kv, axis=-1, keepdims=True)
    kv = kv * jax.lax.rsqrt(var2 + EPS) * kvnw_ref[...]

    cos = cos_ref[...]
    sin = sin_ref[...]
    k1 = kv[:, ND:ND + RD // 2]
    k2 = kv[:, ND + RD // 2:]
    kf_ref[...] = jnp.concatenate(
        [kv[:, :ND], k1 * cos - k2 * sin, k1 * sin + k2 * cos],
        axis=-1).astype(jnp.bfloat16)
    v_ref[...] = kv.astype(jnp.bfloat16)
    for h in range(H):
        qh = q[:, h * HD:(h + 1) * HD]
        q1 = qh[:, ND:ND + RD // 2]
        q2 = qh[:, ND + RD // 2:]
        qf_ref[:, h, :] = jnp.concatenate(
            [qh[:, :ND], q1 * cos - q2 * sin, q1 * sin + q2 * cos],
            axis=-1).astype(jnp.bfloat16)


def _attn_kernel(qf_ref, kf_ref, v_ref, sink_ref, woa_ref, wob_ref, out_ref):
    i = pl.program_id(0)
    qb = i * BQ
    kstart = pl.multiple_of(jnp.maximum(qb - WINDOW, 0), BQ)
    kwin = kf_ref[pl.ds(kstart, KW), :]
    vwin = v_ref[pl.ds(kstart, KW), :]
    rows = qb + jax.lax.broadcasted_iota(jnp.int32, (BQ, KW), 0)
    cols = kstart + jax.lax.broadcasted_iota(jnp.int32, (BQ, KW), 1)
    allowed = (cols <= rows) & (rows - cols < WINDOW)
    sinks = sink_ref[...]
    o_parts = []
    for g in range(G):
        og_parts = []
        for h in range(g * HPG, (g + 1) * HPG):
            qh = qf_ref[:, h, :]
            l = _dot_nt(qh, kwin) * SCALE
            l = jnp.where(allowed, l, NEG)
            m = jnp.max(l, axis=-1, keepdims=True)
            s = sinks[0, h]
            m2 = jnp.maximum(m, s)
            p = jnp.exp(l - m2)
            denom = jnp.sum(p, axis=-1, keepdims=True) + jnp.exp(s - m2)
            oh = _dot_nn(p, vwin)
            og_parts.append(oh / denom)
        og = jnp.concatenate(og_parts, axis=-1)
        woa_g = woa_ref[g, :, :]
        o_parts.append(_dot_nt(og, woa_g))
    o_lat = jnp.concatenate(o_parts, axis=-1)
    out_ref[...] = _dot_nt(o_lat, wob_ref[...])


def kernel(hidden_states, positions, wq_a, q_norm_w, wq_b, wkv, kv_norm_w,
           wo_a, wo_b, attn_sink):
    x = hidden_states.reshape(S, D)

    perm = np.concatenate([np.arange(0, RD, 2), np.arange(1, RD, 2)])
    wqb3 = wq_b.reshape(H, HD, QLR)
    wqb_perm = jnp.concatenate(
        [wqb3[:, :ND, :], wqb3[:, ND:, :][:, perm, :]], axis=1
    ).reshape(H * HD, QLR)
    wkv_perm = jnp.concatenate([wkv[:ND], wkv[ND:][perm]], axis=0)
    kvnw_perm = jnp.concatenate([kv_norm_w[:ND], kv_norm_w[ND:][perm]])
    woa4 = wo_a.reshape(G, OLR, HPG, HD)
    woa_perm = jnp.concatenate(
        [woa4[..., :ND], woa4[..., ND:][..., perm]], axis=-1
    ).reshape(G, OLR, HPG * HD)

    inv_freq = 1.0 / (10000.0 ** (np.arange(0, RD, 2, dtype=np.float32) / RD))
    ang = positions.astype(jnp.float32)[:, None] * inv_freq[None, :]
    cos = jnp.cos(ang)
    sin = jnp.sin(ang)

    full = lambda shape: pl.BlockSpec(shape, lambda i: tuple(0 for _ in shape))
    qf, kf, v = pl.pallas_call(
        _proj_kernel,
        grid=(S // BP,),
        in_specs=[
            pl.BlockSpec((BP, D), lambda i: (i, 0)),
            full((QLR, D)),
            full((1, QLR)),
            full((H * HD, QLR)),
            full((HD, D)),
            full((1, HD)),
            pl.BlockSpec((BP, RD // 2), lambda i: (i, 0)),
            pl.BlockSpec((BP, RD // 2), lambda i: (i, 0)),
        ],
        out_specs=[
            pl.BlockSpec((BP, H, HD), lambda i: (i, 0, 0)),
            pl.BlockSpec((BP, HD), lambda i: (i, 0)),
            pl.BlockSpec((BP, HD), lambda i: (i, 0)),
        ],
        out_shape=[
            jax.ShapeDtypeStruct((S, H, HD), jnp.bfloat16),
            jax.ShapeDtypeStruct((S, HD), jnp.bfloat16),
            jax.ShapeDtypeStruct((S, HD), jnp.bfloat16),
        ],
    )(x, wq_a, q_norm_w.reshape(1, QLR), wqb_perm, wkv_perm,
      kvnw_perm.reshape(1, HD), cos, sin)

    out = pl.pallas_call(
        _attn_kernel,
        grid=(S // BQ,),
        in_specs=[
            pl.BlockSpec((BQ, H, HD), lambda i: (i, 0, 0)),
            full((S, HD)),
            full((S, HD)),
            full((1, H)),
            full((G, OLR, HPG * HD)),
            full((D, G * OLR)),
        ],
        out_specs=pl.BlockSpec((BQ, D), lambda i: (i, 0)),
        out_shape=jax.ShapeDtypeStruct((S, D), jnp.float32),
    )(qf, kf, v, attn_sink.reshape(1, H), woa_perm, wo_b)

    return out.reshape(B, S, D)


# speedup vs baseline: 1.2295x; 1.2295x over previous
import jax
import jax.numpy as jnp
import numpy as np
from jax.experimental import pallas as pl
from jax.experimental.pallas import tpu as pltpu

B, S, D = 1, 2048, 2048
H, HD, RD = 16, 192, 64
ND = HD - RD
QLR, OLR, G = 1024, 128, 4
WINDOW = 512
EPS = 1e-6
SCALE = HD ** -0.5
NEG = -1e30

BP = 256
BQ = 256
KW = WINDOW + BQ
HPG = H // G

_PERM = np.concatenate([np.arange(0, RD, 2), np.arange(1, RD, 2)])
_IDX_HD = np.concatenate([np.arange(ND), ND + _PERM])
_IDX_QROWS = (np.arange(H * HD).reshape(H, HD) // HD * HD +
              _IDX_HD[None, :]).reshape(-1)
_IDX_OCOLS = (np.arange(HPG * HD).reshape(HPG, HD) // HD * HD +
              _IDX_HD[None, :]).reshape(-1)


def _proj_kernel(x_ref, wqa_ref, qnw_ref, wqb_ref, wkv_ref, kvnw_ref,
                 cos_ref, sin_ref, qf_ref, kf_ref, v_ref):
    x = x_ref[...]
    ql = jax.lax.dot_general(x, wqa_ref[...], (((1,), (1,)), ((), ())))
    var = jnp.mean(ql * ql, axis=-1, keepdims=True)
    ql = ql * jax.lax.rsqrt(var + EPS) * qnw_ref[...]
    q = jax.lax.dot_general(ql, wqb_ref[...], (((1,), (1,)), ((), ())))
    q = q * SCALE

    kv = jax.lax.dot_general(x, wkv_ref[...], (((1,), (1,)), ((), ())))
    var2 = jnp.mean(kv * kv, axis=-1, keepdims=True)
    kv = kv * jax.lax.rsqrt(var2 + EPS) * kvnw_ref[...]

    cos = cos_ref[...]
    sin = sin_ref[...]
    k1 = kv[:, ND:ND + RD // 2]
    k2 = kv[:, ND + RD // 2:]
    kf_ref[...] = jnp.concatenate(
        [kv[:, :ND], k1 * cos - k2 * sin, k1 * sin + k2 * cos], axis=-1)
    v_ref[...] = kv
    for h in range(H):
        qh = q[:, h * HD:(h + 1) * HD]
        q1 = qh[:, ND:ND + RD // 2]
        q2 = qh[:, ND + RD // 2:]
        qf_ref[:, h, :] = jnp.concatenate(
            [qh[:, :ND], q1 * cos - q2 * sin, q1 * sin + q2 * cos], axis=-1)


def _attn_kernel(qf_ref, kf_ref, v_ref, sink_ref, woa_ref, wob_ref, out_ref):
    i = pl.program_id(0)
    qb = i * BQ
    kstart = pl.multiple_of(jnp.maximum(qb - WINDOW, 0), BQ)
    kwin = kf_ref[pl.ds(kstart, KW), :]
    vwin = v_ref[pl.ds(kstart, KW), :]
    rows = qb + jax.lax.broadcasted_iota(jnp.int32, (BQ, KW), 0)
    cols = kstart + jax.lax.broadcasted_iota(jnp.int32, (BQ, KW), 1)
    allowed = (cols <= rows) & (rows - cols < WINDOW)
    bias = jnp.where(allowed, 0.0, NEG)
    sinks = sink_ref[...]
    o_parts = []
    for g in range(G):
        og_parts = []
        for h in range(g * HPG, (g + 1) * HPG):
            qh = qf_ref[:, h, :]
            l = jax.lax.dot_general(
                qh, kwin, (((1,), (1,)), ((), ()))) + bias
            m = jnp.max(l, axis=-1, keepdims=True)
            s = sinks[0, h]
            m2 = jnp.maximum(m, s)
            p = jnp.exp(l - m2)
            denom = jnp.sum(p, axis=-1, keepdims=True) + jnp.exp(s - m2)
            oh = jax.lax.dot_general(p, vwin, (((1,), (0,)), ((), ())))
            og_parts.append(oh / denom)
        og = jnp.concatenate(og_parts, axis=-1)
        woa_g = woa_ref[g, :, :]
        o_parts.append(
            jax.lax.dot_general(og, woa_g, (((1,), (1,)), ((), ()))))
    o_lat = jnp.concatenate(o_parts, axis=-1)
    out_ref[...] = jax.lax.dot_general(
        o_lat, wob_ref[...], (((1,), (1,)), ((), ())))


def kernel(hidden_states, positions, wq_a, q_norm_w, wq_b, wkv, kv_norm_w,
           wo_a, wo_b, attn_sink):
    x = hidden_states.reshape(S, D)

    wqb_perm = wq_b[_IDX_QROWS, :]
    wkv_perm = wkv[_IDX_HD, :]
    kvnw_perm = kv_norm_w[_IDX_HD]
    woa_perm = wo_a[:, _IDX_OCOLS].reshape(G, OLR, HPG * HD)

    inv_freq = 1.0 / (10000.0 ** (np.arange(0, RD, 2, dtype=np.float32) / RD))
    ang = positions.astype(jnp.float32)[:, None] * inv_freq[None, :]
    cos = jnp.cos(ang)
    sin = jnp.sin(ang)

    full = lambda shape: pl.BlockSpec(shape, lambda i: tuple(0 for _ in shape))
    qf, kf, v = pl.pallas_call(
        _proj_kernel,
        grid=(S // BP,),
        in_specs=[
            pl.BlockSpec((BP, D), lambda i: (i, 0)),
            full((QLR, D)),
            full((1, QLR)),
            full((H * HD, QLR)),
            full((HD, D)),
            full((1, HD)),
            pl.BlockSpec((BP, RD // 2), lambda i: (i, 0)),
            pl.BlockSpec((BP, RD // 2), lambda i: (i, 0)),
        ],
        out_specs=[
            pl.BlockSpec((BP, H, HD), lambda i: (i, 0, 0)),
            pl.BlockSpec((BP, HD), lambda i: (i, 0)),
            pl.BlockSpec((BP, HD), lambda i: (i, 0)),
        ],
        out_shape=[
            jax.ShapeDtypeStruct((S, H, HD), jnp.float32),
            jax.ShapeDtypeStruct((S, HD), jnp.float32),
            jax.ShapeDtypeStruct((S, HD), jnp.float32),
        ],
    )(x, wq_a, q_norm_w.reshape(1, QLR), wqb_perm, wkv_perm,
      kvnw_perm.reshape(1, HD), cos, sin)

    out = pl.pallas_call(
        _attn_kernel,
        grid=(S // BQ,),
        in_specs=[
            pl.BlockSpec((BQ, H, HD), lambda i: (i, 0, 0)),
            full((S, HD)),
            full((S, HD)),
            full((1, H)),
            full((G, OLR, HPG * HD)),
            full((D, G * OLR)),
        ],
        out_specs=pl.BlockSpec((BQ, D), lambda i: (i, 0)),
        out_shape=jax.ShapeDtypeStruct((S, D), jnp.float32),
    )(qf, kf, v, attn_sink.reshape(1, H), woa_perm, wo_b)

    return out.reshape(B, S, D)


# speedup vs baseline: 1.2749x; 1.0370x over previous
import jax
import jax.numpy as jnp
import numpy as np
from jax.experimental import pallas as pl
from jax.experimental.pallas import tpu as pltpu

B, S, D = 1, 2048, 2048
H, HD, RD = 16, 192, 64
ND = HD - RD
QLR, OLR, G = 1024, 128, 4
WINDOW = 512
EPS = 1e-6
SCALE = HD ** -0.5
NEG = -1e30

BP = 256
BQ = 256
KW = WINDOW + BQ
HPG = H // G

_PERM = np.concatenate([np.arange(0, RD, 2), np.arange(1, RD, 2)])
_IDX_HD = np.concatenate([np.arange(ND), ND + _PERM])
_IDX_QROWS = (np.arange(H * HD).reshape(H, HD) // HD * HD +
              _IDX_HD[None, :]).reshape(-1)
_IDX_OCOLS = (np.arange(HPG * HD).reshape(HPG, HD) // HD * HD +
              _IDX_HD[None, :]).reshape(-1)


def _proj_kernel(x_ref, wqa_ref, qnw_ref, wqb_ref, wkv_ref, kvnw_ref,
                 cos_ref, sin_ref, qf_ref, kf_ref, v_ref):
    x = x_ref[...]
    ql = jax.lax.dot_general(x, wqa_ref[...], (((1,), (1,)), ((), ())))
    var = jnp.mean(ql * ql, axis=-1, keepdims=True)
    ql = ql * jax.lax.rsqrt(var + EPS) * qnw_ref[...]
    q = jax.lax.dot_general(ql, wqb_ref[...], (((1,), (1,)), ((), ())))
    q = q * SCALE

    kv = jax.lax.dot_general(x, wkv_ref[...], (((1,), (1,)), ((), ())))
    var2 = jnp.mean(kv * kv, axis=-1, keepdims=True)
    kv = kv * jax.lax.rsqrt(var2 + EPS) * kvnw_ref[...]

    cos = cos_ref[...]
    sin = sin_ref[...]
    k1 = kv[:, ND:ND + RD // 2]
    k2 = kv[:, ND + RD // 2:]
    kf_ref[...] = jnp.concatenate(
        [kv[:, :ND], k1 * cos - k2 * sin, k1 * sin + k2 * cos], axis=-1)
    v_ref[...] = kv
    for h in range(H):
        qh = q[:, h * HD:(h + 1) * HD]
        q1 = qh[:, ND:ND + RD // 2]
        q2 = qh[:, ND + RD // 2:]
        qf_ref[h, :, :] = jnp.concatenate(
            [qh[:, :ND], q1 * cos - q2 * sin, q1 * sin + q2 * cos], axis=-1)


def _attn_kernel(qf_ref, kf_ref, v_ref, sinkcol_ref, woa_ref, wob_ref,
                 out_ref):
    i = pl.program_id(0)
    qb = i * BQ
    kstart = pl.multiple_of(jnp.maximum(qb - WINDOW, 0), BQ)
    kwin = kf_ref[pl.ds(kstart, KW), :]
    vwin = v_ref[pl.ds(kstart, KW), :]
    rows = qb + jax.lax.broadcasted_iota(jnp.int32, (H * BQ, KW), 0) % BQ
    cols = kstart + jax.lax.broadcasted_iota(jnp.int32, (H * BQ, KW), 1)
    bias = jnp.where((cols <= rows) & (rows - cols < WINDOW), 0.0, NEG)
    s = sinkcol_ref[...]

    qall = qf_ref[...].reshape(H * BQ, HD)
    l = jax.lax.dot_general(qall, kwin, (((1,), (1,)), ((), ()))) + bias
    m = jnp.max(l, axis=-1, keepdims=True)
    m2 = jnp.maximum(m, s)
    p = jnp.exp(l - m2)
    denom = jnp.sum(p, axis=-1, keepdims=True) + jnp.exp(s - m2)
    o = jax.lax.dot_general(p, vwin, (((1,), (0,)), ((), ())))
    o = (o / denom).reshape(H, BQ, HD)

    o_parts = []
    for g in range(G):
        acc = None
        for j in range(HPG):
            h = g * HPG + j
            w = woa_ref[g, :, j * HD:(j + 1) * HD]
            t = jax.lax.dot_general(o[h], w, (((1,), (1,)), ((), ())))
            acc = t if acc is None else acc + t
        o_parts.append(acc)
    o_lat = jnp.concatenate(o_parts, axis=-1)
    out_ref[...] = jax.lax.dot_general(
        o_lat, wob_ref[...], (((1,), (1,)), ((), ())))


def kernel(hidden_states, positions, wq_a, q_norm_w, wq_b, wkv, kv_norm_w,
           wo_a, wo_b, attn_sink):
    x = hidden_states.reshape(S, D)

    wqb_perm = wq_b[_IDX_QROWS, :]
    wkv_perm = wkv[_IDX_HD, :]
    kvnw_perm = kv_norm_w[_IDX_HD]
    woa_perm = wo_a[:, _IDX_OCOLS].reshape(G, OLR, HPG * HD)
    sinkcol = jnp.repeat(attn_sink, BQ)[:, None]

    inv_freq = 1.0 / (10000.0 ** (np.arange(0, RD, 2, dtype=np.float32) / RD))
    ang = positions.astype(jnp.float32)[:, None] * inv_freq[None, :]
    cos = jnp.cos(ang)
    sin = jnp.sin(ang)

    full = lambda shape: pl.BlockSpec(shape, lambda i: tuple(0 for _ in shape))
    qf, kf, v = pl.pallas_call(
        _proj_kernel,
        grid=(S // BP,),
        in_specs=[
            pl.BlockSpec((BP, D), lambda i: (i, 0)),
            full((QLR, D)),
            full((1, QLR)),
            full((H * HD, QLR)),
            full((HD, D)),
            full((1, HD)),
            pl.BlockSpec((BP, RD // 2), lambda i: (i, 0)),
            pl.BlockSpec((BP, RD // 2), lambda i: (i, 0)),
        ],
        out_specs=[
            pl.BlockSpec((H, BP, HD), lambda i: (0, i, 0)),
            pl.BlockSpec((BP, HD), lambda i: (i, 0)),
            pl.BlockSpec((BP, HD), lambda i: (i, 0)),
        ],
        out_shape=[
            jax.ShapeDtypeStruct((H, S, HD), jnp.float32),
            jax.ShapeDtypeStruct((S, HD), jnp.float32),
            jax.ShapeDtypeStruct((S, HD), jnp.float32),
        ],
    )(x, wq_a, q_norm_w.reshape(1, QLR), wqb_perm, wkv_perm,
      kvnw_perm.reshape(1, HD), cos, sin)

    out = pl.pallas_call(
        _attn_kernel,
        grid=(S // BQ,),
        in_specs=[
            pl.BlockSpec((H, BQ, HD), lambda i: (0, i, 0)),
            full((S, HD)),
            full((S, HD)),
            full((H * BQ, 1)),
            full((G, OLR, HPG * HD)),
            full((D, G * OLR)),
        ],
        out_specs=pl.BlockSpec((BQ, D), lambda i: (i, 0)),
        out_shape=jax.ShapeDtypeStruct((S, D), jnp.float32),
    )(qf, kf, v, sinkcol, woa_perm, wo_b)

    return out.reshape(B, S, D)
